# Initial kernel scaffold; baseline (speedup 1.0000x reference)
import jax
import jax.numpy as jnp
from jax.experimental import pallas as pl


def kernel(x, gsp):
    raise NotImplementedError("write your pallas kernel here")


# speedup vs baseline: 1.0141x; 1.0141x over previous
import functools

import jax
import jax.numpy as jnp
from jax.experimental import pallas as pl
from jax.experimental.pallas import tpu as pltpu


def _softmax_kernel(g_ref, w_ref):
    g = g_ref[...].astype(jnp.float32)
    m = jnp.max(g, axis=-1, keepdims=True)
    e = jnp.exp(g - m)
    w_ref[...] = e * pl.reciprocal(jnp.sum(e, axis=-1, keepdims=True),
                                   approx=False)


def _pool_kernel(w_ref, x_ref, o_ref, *, tb):
    w = w_ref[...]
    for b in range(tb):
        xb = x_ref[b]
        o_ref[b, :] = jnp.sum(xb * w, axis=-1)


def kernel(x, gsp):
    B, C, H, W = x.shape
    HW = H * W
    assert gsp.shape == (C, HW)

    x_flat = x.reshape(B, C, HW)
    if x_flat.dtype != jnp.float32:
        x_flat = x_flat.astype(jnp.float32)

    vmem_limit = 64 * 1024 * 1024

    w = pl.pallas_call(
        _softmax_kernel,
        out_shape=jax.ShapeDtypeStruct((C, HW), jnp.float32),
        in_specs=[pl.BlockSpec((C, HW), lambda: (0, 0))],
        out_specs=pl.BlockSpec((C, HW), lambda: (0, 0)),
        compiler_params=pltpu.CompilerParams(vmem_limit_bytes=vmem_limit),
        cost_estimate=pl.CostEstimate(
            flops=4 * C * HW,
            transcendentals=C * HW,
            bytes_accessed=2 * C * HW * 4),
    )(gsp)

    TB = 8
    while B % TB != 0:
        TB //= 2
    nb = B // TB

    pool_fn = functools.partial(_pool_kernel, tb=TB)

    x_bytes = B * C * HW * 4
    return pl.pallas_call(
        pool_fn,
        out_shape=jax.ShapeDtypeStruct((B, C), x.dtype),
        grid=(nb,),
        in_specs=[
            pl.BlockSpec((C, HW), lambda b: (0, 0)),
            pl.BlockSpec((TB, C, HW), lambda b: (b, 0, 0)),
        ],
        out_specs=pl.BlockSpec((TB, C), lambda b: (b, 0)),
        compiler_params=pltpu.CompilerParams(
            dimension_semantics=("parallel",),
            vmem_limit_bytes=vmem_limit),
        cost_estimate=pl.CostEstimate(
            flops=2 * B * C * HW,
            transcendentals=0,
            bytes_accessed=x_bytes + C * HW * 4 + B * C * 4),
    )(w, x_flat)
